# Initial kernel scaffold; baseline (speedup 1.0000x reference)
import jax
import jax.numpy as jnp
from jax.experimental import pallas as pl


def kernel(x_m, x_d, data_m, data_d, edge_index_m, edge_index_d, Wx1, bx1, Wx2, bx2, Wy1, by1, Wy2, by2, Lx1W, Lx1b, Lx2W, Lx2b, Lx3W, Lx3b, Ly1W, Ly1b, Ly2W, Ly2b, Ly3W, Ly3b):
    raise NotImplementedError("write your pallas kernel here")


# speedup vs baseline: 6.2040x; 6.2040x over previous
import functools

import jax
import jax.numpy as jnp
from jax import lax
from jax.experimental import pallas as pl
from jax.experimental.pallas import tpu as pltpu
from jax.experimental.pallas import tpu_sc as plsc

f32 = jnp.float32
i32 = jnp.int32

N = 10000
F = 128
E = 320000
NC, NS, L = 2, 16, 16
CH = 128
NCH = -(-E // (NS * CH))
EP = NS * NCH * CH
SLICE = 640
NR = NS * SLICE

_mesh = plsc.VectorSubcoreMesh(
    core_axis_name="c", subcore_axis_name="s", num_cores=NC, num_subcores=NS)


@functools.partial(
    pl.kernel,
    out_type=(
        jax.ShapeDtypeStruct((NS, NCH, CH), f32),
        jax.ShapeDtypeStruct((NS, NCH, CH), f32),
        jax.ShapeDtypeStruct((NR,), f32),
        jax.ShapeDtypeStruct((NR,), f32),
    ),
    mesh=_mesh,
    scratch_types=(
        pltpu.VMEM((CH,), i32),
        pltpu.VMEM((CH,), i32),
        pltpu.VMEM((CH,), i32),
        pltpu.VMEM((CH,), f32),
        pltpu.VMEM((CH,), f32),
        pltpu.VMEM((CH,), f32),
        pltpu.VMEM((SLICE,), f32),
        pltpu.VMEM_SHARED((NR,), f32),
        pltpu.SemaphoreType.DMA,
    ),
)
def _sc_edge_weights(dm_ref, dd_ref, srcm, dstm, srcd, dstd, valid,
                     wm_out, wd_out, degm_out, degd_out,
                     src_v, dst_v, idx_v, wraw_v, w_v, val_v, zbuf, deg_sp,
                     gsem):
    c = lax.axis_index("c")
    s = lax.axis_index("s")

    @pl.loop(0, SLICE // L)
    def _(i):
        zbuf[pl.ds(i * L, L)] = jnp.zeros((L,), f32)

    pltpu.sync_copy(zbuf, deg_sp.at[pl.ds(s * SLICE, SLICE)])
    plsc.subcore_barrier()

    def run(data_ref, src_ref, dst_ref, w_out):
        @pl.loop(0, NCH)
        def _(j):
            pltpu.sync_copy(src_ref.at[s, j], src_v)
            pltpu.sync_copy(dst_ref.at[s, j], dst_v)
            pltpu.sync_copy(valid.at[s, j], val_v)
            for k in range(CH // L):
                sl = pl.ds(k * L, L)
                idx_v[sl] = src_v[sl] * N + dst_v[sl]
            pltpu.async_copy(data_ref.at[idx_v], wraw_v, gsem).wait()
            for k in range(CH // L):
                sl = pl.ds(k * L, L)
                w_v[sl] = jnp.maximum(wraw_v[sl], 0.0) * val_v[sl]
            pltpu.sync_copy(w_v, w_out.at[s, j])
            pltpu.sync_copy(w_v, deg_sp.at[dst_v], add=True)

    @pl.when(c == 0)
    def _():
        run(dm_ref, srcm, dstm, wm_out)

    @pl.when(c == 1)
    def _():
        run(dd_ref, srcd, dstd, wd_out)

    plsc.subcore_barrier()

    @pl.when(c == 0)
    def _():
        pltpu.sync_copy(deg_sp.at[pl.ds(s * SLICE, SLICE)],
                        degm_out.at[pl.ds(s * SLICE, SLICE)])

    @pl.when(c == 1)
    def _():
        pltpu.sync_copy(deg_sp.at[pl.ds(s * SLICE, SLICE)],
                        degd_out.at[pl.ds(s * SLICE, SLICE)])


@functools.partial(
    pl.kernel,
    out_type=(
        jax.ShapeDtypeStruct((NR, F), f32),
        jax.ShapeDtypeStruct((NR, F), f32),
    ),
    mesh=_mesh,
    scratch_types=(
        pltpu.VMEM((CH,), i32),
        pltpu.VMEM((CH,), i32),
        pltpu.VMEM((CH,), f32),
        pltpu.VMEM((CH, F), f32),
        pltpu.VMEM_SHARED((NR, F), f32),
        pltpu.SemaphoreType.DMA,
    ),
)
def _sc_msg(um_ref, ud_ref, wm_ref, wd_ref, srcm, dstm, srcd, dstd,
            aggm_out, aggd_out, sidx, didx, w_v, rows, acc, gsem):
    c = lax.axis_index("c")
    s = lax.axis_index("s")

    @pl.loop(0, CH)
    def _(r):
        for f in range(F // L):
            rows[r, pl.ds(f * L, L)] = jnp.zeros((L,), f32)

    for t in range(SLICE // CH):
        pltpu.sync_copy(rows, acc.at[pl.ds(s * SLICE + t * CH, CH)])
    plsc.subcore_barrier()

    def run(u_ref, w_ref, src_ref, dst_ref):
        @pl.loop(0, NCH)
        def _(j):
            pltpu.sync_copy(src_ref.at[s, j], sidx)
            pltpu.async_copy(u_ref.at[sidx], rows, gsem).wait()
            pltpu.sync_copy(w_ref.at[s, j], w_v)
            pltpu.sync_copy(dst_ref.at[s, j], didx)

            @pl.loop(0, CH // L)
            def _(r16):
                wv = w_v[pl.ds(r16 * L, L)]
                for lane in range(L):
                    r = r16 * L + lane
                    ws = wv[lane]
                    for f in range(F // L):
                        sl = pl.ds(f * L, L)
                        rows[r, sl] = rows[r, sl] * ws

            pltpu.sync_copy(rows, acc.at[didx], add=True)

    @pl.when(c == 0)
    def _():
        run(um_ref, wm_ref, srcm, dstm)

    @pl.when(c == 1)
    def _():
        run(ud_ref, wd_ref, srcd, dstd)

    plsc.subcore_barrier()

    def out(agg_out):
        for t in range(SLICE // CH):
            off = s * SLICE + t * CH
            pltpu.sync_copy(acc.at[pl.ds(off, CH)], agg_out.at[pl.ds(off, CH)])

    @pl.when(c == 0)
    def _():
        out(aggm_out)

    @pl.when(c == 1)
    def _():
        out(aggd_out)


_RB = 2000


def _relu(x):
    return jnp.maximum(x, 0.0)


def _tc_prep_body(degm, degd, xm, xd, wm, wd, dinvm_o, dinvd_o, um_o, ud_o):
    dm = lax.rsqrt(degm[...] + 1.0)
    dd = lax.rsqrt(degd[...] + 1.0)
    dinvm_o[...] = dm
    dinvd_o[...] = dd
    um_o[...] = jnp.dot(xm[...], wm[...], preferred_element_type=f32) * dm
    ud_o[...] = jnp.dot(xd[...], wd[...], preferred_element_type=f32) * dd


_col = lambda i: (i, 0)
_fix = lambda i: (0, 0)

_tc_prep = pl.pallas_call(
    _tc_prep_body,
    grid=(N // _RB,),
    in_specs=[
        pl.BlockSpec((_RB, 1), _col), pl.BlockSpec((_RB, 1), _col),
        pl.BlockSpec((_RB, F), _col), pl.BlockSpec((_RB, F), _col),
        pl.BlockSpec((F, F), _fix), pl.BlockSpec((F, F), _fix),
    ],
    out_specs=[
        pl.BlockSpec((_RB, 1), _col), pl.BlockSpec((_RB, 1), _col),
        pl.BlockSpec((_RB, F), _col), pl.BlockSpec((_RB, F), _col),
    ],
    out_shape=[
        jax.ShapeDtypeStruct((N, 1), f32), jax.ShapeDtypeStruct((N, 1), f32),
        jax.ShapeDtypeStruct((N, F), f32), jax.ShapeDtypeStruct((N, F), f32),
    ],
)


def _tc_mid_body(aggm, aggd, um, ud, dinvm, dinvd, bm, bd, wm, wd,
                 u2m_o, u2d_o):
    x1m = _relu((aggm[...] + um[...]) * dinvm[...] + bm[...])
    x1d = _relu((aggd[...] + ud[...]) * dinvd[...] + bd[...])
    u2m_o[...] = jnp.dot(x1m, wm[...], preferred_element_type=f32) * dinvm[...]
    u2d_o[...] = jnp.dot(x1d, wd[...], preferred_element_type=f32) * dinvd[...]


_tc_mid = pl.pallas_call(
    _tc_mid_body,
    grid=(N // _RB,),
    in_specs=[
        pl.BlockSpec((_RB, F), _col), pl.BlockSpec((_RB, F), _col),
        pl.BlockSpec((_RB, F), _col), pl.BlockSpec((_RB, F), _col),
        pl.BlockSpec((_RB, 1), _col), pl.BlockSpec((_RB, 1), _col),
        pl.BlockSpec((1, F), _fix), pl.BlockSpec((1, F), _fix),
        pl.BlockSpec((F, F), _fix), pl.BlockSpec((F, F), _fix),
    ],
    out_specs=[
        pl.BlockSpec((_RB, F), _col), pl.BlockSpec((_RB, F), _col),
    ],
    out_shape=[
        jax.ShapeDtypeStruct((N, F), f32), jax.ShapeDtypeStruct((N, F), f32),
    ],
)


def _tc_tail_body(aggm, aggd, um, ud, dinvm, dinvd, bm, bd,
                  l1wm, l1bm, l2wm, l2bm, l3wm, l3bm,
                  l1wd, l1bd, l2wd, l2bd, l3wd, l3bd,
                  xfm_o, xfd_o):
    def mlp(agg, u, dinv, b, l1w, l1b, l2w, l2b, l3w, l3b):
        x = _relu((agg[...] + u[...]) * dinv[...] + b[...])
        h = _relu(jnp.dot(x, l1w[...], preferred_element_type=f32) + l1b[...])
        h = _relu(jnp.dot(h, l2w[...], preferred_element_type=f32) + l2b[...])
        return _relu(jnp.dot(h, l3w[...], preferred_element_type=f32) + l3b[...])

    xfm_o[...] = mlp(aggm, um, dinvm, bm, l1wm, l1bm, l2wm, l2bm, l3wm, l3bm)
    xfd_o[...] = mlp(aggd, ud, dinvd, bd, l1wd, l1bd, l2wd, l2bd, l3wd, l3bd)


_tc_tail = pl.pallas_call(
    _tc_tail_body,
    grid=(N // _RB,),
    in_specs=[
        pl.BlockSpec((_RB, F), _col), pl.BlockSpec((_RB, F), _col),
        pl.BlockSpec((_RB, F), _col), pl.BlockSpec((_RB, F), _col),
        pl.BlockSpec((_RB, 1), _col), pl.BlockSpec((_RB, 1), _col),
        pl.BlockSpec((1, F), _fix), pl.BlockSpec((1, F), _fix),
        pl.BlockSpec((F, 256), _fix), pl.BlockSpec((1, 256), _fix),
        pl.BlockSpec((256, 128), _fix), pl.BlockSpec((1, 128), _fix),
        pl.BlockSpec((128, 64), _fix), pl.BlockSpec((1, 64), _fix),
        pl.BlockSpec((F, 256), _fix), pl.BlockSpec((1, 256), _fix),
        pl.BlockSpec((256, 128), _fix), pl.BlockSpec((1, 128), _fix),
        pl.BlockSpec((128, 64), _fix), pl.BlockSpec((1, 64), _fix),
    ],
    out_specs=[
        pl.BlockSpec((_RB, 64), _col), pl.BlockSpec((_RB, 64), _col),
    ],
    out_shape=[
        jax.ShapeDtypeStruct((N, 64), f32), jax.ShapeDtypeStruct((N, 64), f32),
    ],
)

_OB = 400


def _tc_outer_body(a, b, o):
    o[...] = lax.dot_general(a[...], b[...], (((1,), (1,)), ((), ())),
                             preferred_element_type=f32)


_tc_outer = pl.pallas_call(
    _tc_outer_body,
    grid=(N // _OB,),
    in_specs=[
        pl.BlockSpec((_OB, 64), lambda i: (i, 0)),
        pl.BlockSpec((N, 64), lambda i: (0, 0)),
    ],
    out_specs=pl.BlockSpec((_OB, N), lambda i: (i, 0)),
    out_shape=jax.ShapeDtypeStruct((N, N), f32),
)


def kernel(x_m, x_d, data_m, data_d, edge_index_m, edge_index_d,
           Wx1, bx1, Wx2, bx2, Wy1, by1, Wy2, by2,
           Lx1W, Lx1b, Lx2W, Lx2b, Lx3W, Lx3b,
           Ly1W, Ly1b, Ly2W, Ly2b, Ly3W, Ly3b):
    pad = EP - E
    fill = (jnp.arange(pad, dtype=i32) * 37) % N

    def prep(ei):
        src = jnp.concatenate([ei[0], fill]).reshape(NS, NCH, CH)
        dst = jnp.concatenate([ei[1], fill]).reshape(NS, NCH, CH)
        return src, dst

    src_m, dst_m = prep(edge_index_m)
    src_d, dst_d = prep(edge_index_d)
    valid = jnp.concatenate(
        [jnp.ones((E,), f32), jnp.zeros((pad,), f32)]).reshape(NS, NCH, CH)

    w_m, w_d, deg_m, deg_d = _sc_edge_weights(
        data_m.reshape(-1), data_d.reshape(-1),
        src_m, dst_m, src_d, dst_d, valid)

    dinv_m, dinv_d, u1_m, u1_d = _tc_prep(
        deg_m[:N].reshape(N, 1), deg_d[:N].reshape(N, 1), x_m, x_d, Wx1, Wy1)

    agg1_m, agg1_d = _sc_msg(u1_m, u1_d, w_m, w_d,
                             src_m, dst_m, src_d, dst_d)

    u2_m, u2_d = _tc_mid(agg1_m[:N], agg1_d[:N], u1_m, u1_d, dinv_m, dinv_d,
                         bx1.reshape(1, F), by1.reshape(1, F), Wx2, Wy2)

    agg2_m, agg2_d = _sc_msg(u2_m, u2_d, w_m, w_d,
                             src_m, dst_m, src_d, dst_d)

    xf_m, xf_d = _tc_tail(
        agg2_m[:N], agg2_d[:N], u2_m, u2_d, dinv_m, dinv_d,
        bx2.reshape(1, F), by2.reshape(1, F),
        Lx1W, Lx1b.reshape(1, 256), Lx2W, Lx2b.reshape(1, 128),
        Lx3W, Lx3b.reshape(1, 64),
        Ly1W, Ly1b.reshape(1, 256), Ly2W, Ly2b.reshape(1, 128),
        Ly3W, Ly3b.reshape(1, 64))

    return _tc_outer(xf_m, xf_d)


# speedup vs baseline: 11.0450x; 1.7803x over previous
import functools

import jax
import jax.numpy as jnp
from jax import lax
from jax.experimental import pallas as pl
from jax.experimental.pallas import tpu as pltpu
from jax.experimental.pallas import tpu_sc as plsc

f32 = jnp.float32
i32 = jnp.int32

N = 10000
F = 128
E = 320000
NC, NS, L = 2, 16, 16
CH = 112
NCH = 180
NCHX = NCH + 4
EP = NS * NCH * CH
EPT = NCH * CH
SLICE = 640
NR = NS * SLICE

_mesh = plsc.VectorSubcoreMesh(
    core_axis_name="c", subcore_axis_name="s", num_cores=NC, num_subcores=NS)


@functools.partial(
    pl.kernel,
    out_type=(
        jax.ShapeDtypeStruct((NS, NCHX, CH), f32),
        jax.ShapeDtypeStruct((NS, NCHX, CH), f32),
        jax.ShapeDtypeStruct((NR,), f32),
        jax.ShapeDtypeStruct((NR,), f32),
    ),
    mesh=_mesh,
    scratch_types=(
        pltpu.VMEM((NCHX, CH), i32),
        pltpu.VMEM((NCHX, CH), i32),
        pltpu.VMEM((NCHX, CH), f32),
        pltpu.VMEM((3, CH), i32),
        pltpu.VMEM((3, CH), f32),
        pltpu.VMEM((SLICE,), f32),
        pltpu.VMEM_SHARED((NR,), f32),
        pltpu.SemaphoreType.DMA,
        pltpu.SemaphoreType.DMA,
        pltpu.SemaphoreType.DMA,
    ),
)
def _sc_edge_weights(dm_ref, dd_ref, srcm, dstm, srcd, dstd,
                     wm_out, wd_out, degm_out, degd_out,
                     srcall, dstall, wres, idxb, wraw, zbuf, deg_sp,
                     g0, g1, g2):
    c = lax.axis_index("c")
    s = lax.axis_index("s")
    gsems = (g0, g1, g2)

    @pl.loop(0, SLICE // L)
    def _(i):
        zbuf[pl.ds(i * L, L)] = jnp.zeros((L,), f32)

    pltpu.sync_copy(zbuf, deg_sp.at[pl.ds(s * SLICE, SLICE)])
    plsc.subcore_barrier()

    def run(data_ref, src_ref, dst_ref, w_out):
        pltpu.sync_copy(src_ref.at[s], srcall)
        pltpu.sync_copy(dst_ref.at[s], dstall)

        def gather(j, k):
            for g in range(CH // L):
                sl = pl.ds(g * L, L)
                idxb[k, sl] = srcall[j, sl] * N + dstall[j, sl]
            pltpu.async_copy(data_ref.at[idxb.at[k]], wraw.at[k], gsems[k])

        def gather_wait(k):
            pltpu.make_async_copy(
                data_ref.at[idxb.at[k]], wraw.at[k], gsems[k]).wait()

        def process(j, k):
            gather_wait(k)
            base = s * EPT + j * CH
            for g in range(CH // L):
                sl = pl.ds(g * L, L)
                pos = base + g * L + lax.iota(i32, L)
                w = jnp.maximum(wraw[k, sl], 0.0)
                wres[j, sl] = jnp.where(pos < E, w, 0.0)
            pltpu.sync_copy(wres.at[j], deg_sp.at[dstall.at[j]], add=True)

        gather(0, 0)
        gather(1, 1)

        @pl.loop(0, NCH // 3)
        def _(i):
            for k in range(3):
                j = i * 3 + k
                gather(j + 2, (k + 2) % 3)
                process(j, k)

        gather_wait(0)
        gather_wait(1)
        pltpu.sync_copy(wres, w_out.at[s])

    @pl.when(c == 0)
    def _():
        run(dm_ref, srcm, dstm, wm_out)

    @pl.when(c == 1)
    def _():
        run(dd_ref, srcd, dstd, wd_out)

    plsc.subcore_barrier()

    @pl.when(c == 0)
    def _():
        pltpu.sync_copy(deg_sp.at[pl.ds(s * SLICE, SLICE)],
                        degm_out.at[pl.ds(s * SLICE, SLICE)])

    @pl.when(c == 1)
    def _():
        pltpu.sync_copy(deg_sp.at[pl.ds(s * SLICE, SLICE)],
                        degd_out.at[pl.ds(s * SLICE, SLICE)])


@functools.partial(
    pl.kernel,
    out_type=(
        jax.ShapeDtypeStruct((NR, F), f32),
        jax.ShapeDtypeStruct((NR, F), f32),
    ),
    mesh=_mesh,
    scratch_types=(
        pltpu.VMEM((6, CH), i32),
        pltpu.VMEM((6, CH), i32),
        pltpu.VMEM((6, CH), f32),
        pltpu.VMEM((3, CH, F), f32),
        pltpu.VMEM_SHARED((NR, F), f32),
        pltpu.SemaphoreType.DMA,
        pltpu.SemaphoreType.DMA,
        pltpu.SemaphoreType.DMA,
        pltpu.SemaphoreType.DMA,
        pltpu.SemaphoreType.DMA,
        pltpu.SemaphoreType.DMA,
        pltpu.SemaphoreType.DMA,
        pltpu.SemaphoreType.DMA,
        pltpu.SemaphoreType.DMA,
        pltpu.SemaphoreType.DMA,
        pltpu.SemaphoreType.DMA,
        pltpu.SemaphoreType.DMA,
    ),
)
def _sc_msg(um_ref, ud_ref, wm_ref, wd_ref, srcm, dstm, srcd, dstd,
            aggm_out, aggd_out, sidx6, didx6, w6, rows, acc,
            g0, g1, g2, s0, s1, s2, i0, i1, i2, i3, i4, i5):
    c = lax.axis_index("c")
    s = lax.axis_index("s")
    gsems = (g0, g1, g2)
    ssems = (s0, s1, s2)
    isems = (i0, i1, i2, i3, i4, i5)

    @pl.loop(0, CH)
    def _(r):
        for f in range(F // L):
            rows[0, r, pl.ds(f * L, L)] = jnp.zeros((L,), f32)

    for t in range(-(-SLICE // CH)):
        off = s * SLICE + t * CH
        n = min(CH, SLICE - t * CH)
        pltpu.sync_copy(rows.at[0, pl.ds(0, n)], acc.at[pl.ds(off, n)])
    plsc.subcore_barrier()

    def run(u_ref, w_ref, src_ref, dst_ref):
        def idx_fetch(j, b):
            pltpu.async_copy(src_ref.at[s, j], sidx6.at[b], isems[b])
            pltpu.async_copy(dst_ref.at[s, j], didx6.at[b], isems[b])
            pltpu.async_copy(w_ref.at[s, j], w6.at[b], isems[b])

        def idx_wait(j, b):
            pltpu.make_async_copy(src_ref.at[s, j], sidx6.at[b], isems[b]).wait()
            pltpu.make_async_copy(dst_ref.at[s, j], didx6.at[b], isems[b]).wait()
            pltpu.make_async_copy(w_ref.at[s, j], w6.at[b], isems[b]).wait()

        def gather(k3, b):
            pltpu.async_copy(u_ref.at[sidx6.at[b]], rows.at[k3], gsems[k3])

        def gather_wait(k3, b):
            pltpu.make_async_copy(
                u_ref.at[sidx6.at[b]], rows.at[k3], gsems[k3]).wait()

        def scatter(k3, b):
            pltpu.async_copy(
                rows.at[k3], acc.at[didx6.at[b]], ssems[k3], add=True)

        def scatter_wait(k3, b):
            pltpu.make_async_copy(
                rows.at[k3], acc.at[didx6.at[b]], ssems[k3]).wait()

        def scale(k3, b):
            @pl.loop(0, CH // L)
            def _(r16):
                wv = w6[b, pl.ds(r16 * L, L)]
                for lane in range(L):
                    r = r16 * L + lane
                    ws = wv[lane]
                    for f in range(F // L):
                        sl = pl.ds(f * L, L)
                        rows[k3, r, sl] = rows[k3, r, sl] * ws

        for j in range(4):
            idx_fetch(j, j)
        for j in range(2):
            idx_wait(j, j)
            gather(j % 3, j)

        @pl.loop(0, NCH // 6)
        def _(i):
            for k in range(6):
                j = i * 6 + k
                k3, k3n = k % 3, (k + 2) % 3
                b, bn, bf, bp = k, (k + 2) % 6, (k + 4) % 6, (k + 5) % 6
                idx_fetch(j + 4, bf)
                gather_wait(k3, b)
                scale(k3, b)
                if k == 0:
                    @pl.when(i > 0)
                    def _():
                        scatter_wait(k3n, bp)
                else:
                    scatter_wait(k3n, bp)
                idx_wait(j + 2, bn)
                gather(k3n, bn)
                scatter(k3, b)

        gather_wait(0, 0)
        gather_wait(1, 1)
        scatter_wait(2, 5)
        idx_wait(NCH + 2, 2)
        idx_wait(NCH + 3, 3)

    @pl.when(c == 0)
    def _():
        run(um_ref, wm_ref, srcm, dstm)

    @pl.when(c == 1)
    def _():
        run(ud_ref, wd_ref, srcd, dstd)

    plsc.subcore_barrier()

    def out(agg_out):
        for t in range(-(-SLICE // CH)):
            off = s * SLICE + t * CH
            n = min(CH, SLICE - t * CH)
            pltpu.sync_copy(acc.at[pl.ds(off, n)], agg_out.at[pl.ds(off, n)])

    @pl.when(c == 0)
    def _():
        out(aggm_out)

    @pl.when(c == 1)
    def _():
        out(aggd_out)


_RB = 2000


def _relu(x):
    return jnp.maximum(x, 0.0)


def _tc_prep_body(degm, degd, xm, xd, wm, wd, dinvm_o, dinvd_o, um_o, ud_o):
    dm = lax.rsqrt(degm[...] + 1.0)
    dd = lax.rsqrt(degd[...] + 1.0)
    dinvm_o[...] = dm
    dinvd_o[...] = dd
    um_o[...] = jnp.dot(xm[...], wm[...], preferred_element_type=f32) * dm
    ud_o[...] = jnp.dot(xd[...], wd[...], preferred_element_type=f32) * dd


_col = lambda i: (i, 0)
_fix = lambda i: (0, 0)

_tc_prep = pl.pallas_call(
    _tc_prep_body,
    grid=(N // _RB,),
    in_specs=[
        pl.BlockSpec((_RB, 1), _col), pl.BlockSpec((_RB, 1), _col),
        pl.BlockSpec((_RB, F), _col), pl.BlockSpec((_RB, F), _col),
        pl.BlockSpec((F, F), _fix), pl.BlockSpec((F, F), _fix),
    ],
    out_specs=[
        pl.BlockSpec((_RB, 1), _col), pl.BlockSpec((_RB, 1), _col),
        pl.BlockSpec((_RB, F), _col), pl.BlockSpec((_RB, F), _col),
    ],
    out_shape=[
        jax.ShapeDtypeStruct((N, 1), f32), jax.ShapeDtypeStruct((N, 1), f32),
        jax.ShapeDtypeStruct((N, F), f32), jax.ShapeDtypeStruct((N, F), f32),
    ],
)


def _tc_mid_body(aggm, aggd, um, ud, dinvm, dinvd, bm, bd, wm, wd,
                 u2m_o, u2d_o):
    x1m = _relu((aggm[...] + um[...]) * dinvm[...] + bm[...])
    x1d = _relu((aggd[...] + ud[...]) * dinvd[...] + bd[...])
    u2m_o[...] = jnp.dot(x1m, wm[...], preferred_element_type=f32) * dinvm[...]
    u2d_o[...] = jnp.dot(x1d, wd[...], preferred_element_type=f32) * dinvd[...]


_tc_mid = pl.pallas_call(
    _tc_mid_body,
    grid=(N // _RB,),
    in_specs=[
        pl.BlockSpec((_RB, F), _col), pl.BlockSpec((_RB, F), _col),
        pl.BlockSpec((_RB, F), _col), pl.BlockSpec((_RB, F), _col),
        pl.BlockSpec((_RB, 1), _col), pl.BlockSpec((_RB, 1), _col),
        pl.BlockSpec((1, F), _fix), pl.BlockSpec((1, F), _fix),
        pl.BlockSpec((F, F), _fix), pl.BlockSpec((F, F), _fix),
    ],
    out_specs=[
        pl.BlockSpec((_RB, F), _col), pl.BlockSpec((_RB, F), _col),
    ],
    out_shape=[
        jax.ShapeDtypeStruct((N, F), f32), jax.ShapeDtypeStruct((N, F), f32),
    ],
)


def _tc_tail_body(aggm, aggd, um, ud, dinvm, dinvd, bm, bd,
                  l1wm, l1bm, l2wm, l2bm, l3wm, l3bm,
                  l1wd, l1bd, l2wd, l2bd, l3wd, l3bd,
                  xfm_o, xfd_o):
    def mlp(agg, u, dinv, b, l1w, l1b, l2w, l2b, l3w, l3b):
        x = _relu((agg[...] + u[...]) * dinv[...] + b[...])
        h = _relu(jnp.dot(x, l1w[...], preferred_element_type=f32) + l1b[...])
        h = _relu(jnp.dot(h, l2w[...], preferred_element_type=f32) + l2b[...])
        return _relu(jnp.dot(h, l3w[...], preferred_element_type=f32) + l3b[...])

    xfm_o[...] = mlp(aggm, um, dinvm, bm, l1wm, l1bm, l2wm, l2bm, l3wm, l3bm)
    xfd_o[...] = mlp(aggd, ud, dinvd, bd, l1wd, l1bd, l2wd, l2bd, l3wd, l3bd)


_tc_tail = pl.pallas_call(
    _tc_tail_body,
    grid=(N // _RB,),
    in_specs=[
        pl.BlockSpec((_RB, F), _col), pl.BlockSpec((_RB, F), _col),
        pl.BlockSpec((_RB, F), _col), pl.BlockSpec((_RB, F), _col),
        pl.BlockSpec((_RB, 1), _col), pl.BlockSpec((_RB, 1), _col),
        pl.BlockSpec((1, F), _fix), pl.BlockSpec((1, F), _fix),
        pl.BlockSpec((F, 256), _fix), pl.BlockSpec((1, 256), _fix),
        pl.BlockSpec((256, 128), _fix), pl.BlockSpec((1, 128), _fix),
        pl.BlockSpec((128, 64), _fix), pl.BlockSpec((1, 64), _fix),
        pl.BlockSpec((F, 256), _fix), pl.BlockSpec((1, 256), _fix),
        pl.BlockSpec((256, 128), _fix), pl.BlockSpec((1, 128), _fix),
        pl.BlockSpec((128, 64), _fix), pl.BlockSpec((1, 64), _fix),
    ],
    out_specs=[
        pl.BlockSpec((_RB, 64), _col), pl.BlockSpec((_RB, 64), _col),
    ],
    out_shape=[
        jax.ShapeDtypeStruct((N, 64), f32), jax.ShapeDtypeStruct((N, 64), f32),
    ],
)

_OB = 400


def _tc_outer_body(a, b, o):
    o[...] = lax.dot_general(a[...], b[...], (((1,), (1,)), ((), ())),
                             preferred_element_type=f32)


_tc_outer = pl.pallas_call(
    _tc_outer_body,
    grid=(N // _OB,),
    in_specs=[
        pl.BlockSpec((_OB, 64), lambda i: (i, 0)),
        pl.BlockSpec((N, 64), lambda i: (0, 0)),
    ],
    out_specs=pl.BlockSpec((_OB, N), lambda i: (i, 0)),
    out_shape=jax.ShapeDtypeStruct((N, N), f32),
)


def kernel(x_m, x_d, data_m, data_d, edge_index_m, edge_index_d,
           Wx1, bx1, Wx2, bx2, Wy1, by1, Wy2, by2,
           Lx1W, Lx1b, Lx2W, Lx2b, Lx3W, Lx3b,
           Ly1W, Ly1b, Ly2W, Ly2b, Ly3W, Ly3b):
    pad = EP - E
    fill = (jnp.arange(pad, dtype=i32) * 37) % N
    dummy = ((jnp.arange(NS * 4 * CH, dtype=i32) * 131) % N).reshape(NS, 4, CH)

    def prep(ei):
        src = jnp.concatenate([ei[0], fill]).reshape(NS, NCH, CH)
        dst = jnp.concatenate([ei[1], fill]).reshape(NS, NCH, CH)
        return (jnp.concatenate([src, dummy], axis=1),
                jnp.concatenate([dst, dummy], axis=1))

    src_m, dst_m = prep(edge_index_m)
    src_d, dst_d = prep(edge_index_d)

    w_m, w_d, deg_m, deg_d = _sc_edge_weights(
        data_m.reshape(-1), data_d.reshape(-1),
        src_m, dst_m, src_d, dst_d)

    dinv_m, dinv_d, u1_m, u1_d = _tc_prep(
        deg_m[:N].reshape(N, 1), deg_d[:N].reshape(N, 1), x_m, x_d, Wx1, Wy1)

    agg1_m, agg1_d = _sc_msg(u1_m, u1_d, w_m, w_d,
                             src_m, dst_m, src_d, dst_d)

    u2_m, u2_d = _tc_mid(agg1_m, agg1_d, u1_m, u1_d, dinv_m, dinv_d,
                         bx1.reshape(1, F), by1.reshape(1, F), Wx2, Wy2)

    agg2_m, agg2_d = _sc_msg(u2_m, u2_d, w_m, w_d,
                             src_m, dst_m, src_d, dst_d)

    xf_m, xf_d = _tc_tail(
        agg2_m, agg2_d, u2_m, u2_d, dinv_m, dinv_d,
        bx2.reshape(1, F), by2.reshape(1, F),
        Lx1W, Lx1b.reshape(1, 256), Lx2W, Lx2b.reshape(1, 128),
        Lx3W, Lx3b.reshape(1, 64),
        Ly1W, Ly1b.reshape(1, 256), Ly2W, Ly2b.reshape(1, 128),
        Ly3W, Ly3b.reshape(1, 64))

    return _tc_outer(xf_m, xf_d)
